# baseline (speedup 1.0000x reference)
import jax
import jax.numpy as jnp
from jax.experimental import pallas as pl
from jax.experimental.pallas import tpu as pltpu

INPUT_DIM = 768
HIDDEN = INPUT_DIM // 2
NUM_EXPERTS = 64
BT = 1024


def _gate_kernel(x_ref, w1_ref, b1_ref, w2_ref, b2_ref, out_ref):
    h = jnp.dot(x_ref[:], w1_ref[:], preferred_element_type=jnp.float32)
    h = jnp.maximum(h + b1_ref[:], 0.0)
    logits = jnp.dot(h, w2_ref[:], preferred_element_type=jnp.float32)
    logits = logits + b2_ref[:]

    lane_f = jax.lax.broadcasted_iota(jnp.int32, logits.shape, 1).astype(
        jnp.float32)
    m1 = jnp.max(logits, axis=-1, keepdims=True)
    i1f = jnp.min(jnp.where(logits == m1, lane_f, float(NUM_EXPERTS)),
                  axis=-1, keepdims=True)
    masked = jnp.where(lane_f == i1f, -jnp.inf, logits)
    m2 = jnp.max(masked, axis=-1, keepdims=True)
    i2f = jnp.min(jnp.where(masked == m2, lane_f, float(NUM_EXPERTS)),
                  axis=-1, keepdims=True)

    e = jnp.exp(m2 - m1)
    inv = 1.0 / (1.0 + e)
    out = jnp.concatenate([inv, e * inv, i1f, i2f], axis=1)
    out_ref[:] = out.T


@jax.jit
def kernel(x, W1, b1, W2, b2):
    n = x.shape[0]
    out = pl.pallas_call(
        _gate_kernel,
        grid=(n // BT,),
        in_specs=[
            pl.BlockSpec((BT, INPUT_DIM), lambda i: (i, 0)),
            pl.BlockSpec((INPUT_DIM, HIDDEN), lambda i: (0, 0)),
            pl.BlockSpec((1, HIDDEN), lambda i: (0, 0)),
            pl.BlockSpec((HIDDEN, NUM_EXPERTS), lambda i: (0, 0)),
            pl.BlockSpec((1, NUM_EXPERTS), lambda i: (0, 0)),
        ],
        out_specs=pl.BlockSpec((4, BT), lambda i: (0, i)),
        out_shape=jax.ShapeDtypeStruct((4, n), jnp.float32),
        compiler_params=pltpu.CompilerParams(
            dimension_semantics=("parallel",),
        ),
    )(x, W1, b1.reshape(1, HIDDEN), W2, b2.reshape(1, NUM_EXPERTS))
    return (out[:2].T, out[2:].T.astype(jnp.int32))
